# baseline (speedup 1.0000x reference)
import functools

import jax
import jax.numpy as jnp
from jax import lax
from jax.experimental import pallas as pl
from jax.experimental.pallas import tpu as pltpu
from jax.experimental.pallas import tpu_sc as plsc

_RND = 16


def kernel(pairs, init_emb, output_vecs):
    B = pairs.shape[0]
    V, D = init_emb.shape
    info = plsc.get_sparse_core_info()
    nc, ns = info.num_cores, info.num_subcores
    nw = nc * ns
    b_per_w = B // nw
    n_rounds = b_per_w // _RND

    src_t3 = init_emb.T.reshape(2, 8, V)
    dst_t3 = output_vecs.T.reshape(2, 8, V)

    src_idx = pairs[:, 0].astype(jnp.int32).reshape(nw, b_per_w)
    dst_idx = pairs[:, 1].astype(jnp.int32).reshape(nw, b_per_w)

    mesh = plsc.VectorSubcoreMesh(core_axis_name="c", subcore_axis_name="s")

    @functools.partial(
        pl.kernel,
        mesh=mesh,
        out_type=jax.ShapeDtypeStruct((B,), jnp.float32),
        compiler_params=pltpu.CompilerParams(needs_layout_passes=False),
        scratch_types=[
            pltpu.VMEM((b_per_w,), jnp.int32),
            pltpu.VMEM((b_per_w,), jnp.int32),
            pltpu.VMEM((2, 8, _RND * 128), jnp.float32),
            pltpu.VMEM((2, 8, _RND * 128), jnp.float32),
            pltpu.VMEM((16 * 16,), jnp.float32),
            pltpu.VMEM((b_per_w,), jnp.float32),
            pltpu.SemaphoreType.DMA,
        ],
    )
    def run(src_idx_hbm, dst_idx_hbm, src_t3_hbm, dst_t3_hbm, out_hbm,
            sidx_v, didx_v, sblk_v, dblk_v, prod_v, out_v, sem):
        wid = lax.axis_index("s") * nc + lax.axis_index("c")

        pltpu.sync_copy(src_idx_hbm.at[wid], sidx_v)
        pltpu.sync_copy(dst_idx_hbm.at[wid], didx_v)

        iota16 = lax.iota(jnp.int32, 16)
        cb_v = iota16 // 8
        s_v = iota16 % 8

        def body(g, _):
            base = g * _RND
            siv = sidx_v[pl.ds(base, _RND)]
            div = didx_v[pl.ds(base, _RND)]
            qs, qd = [], []
            for j in range(_RND):
                r = jnp.squeeze(lax.slice(siv, (j,), (j + 1,)))
                r2 = jnp.squeeze(lax.slice(div, (j,), (j + 1,)))
                qs.append(r % 128)
                qd.append(r2 % 128)
                rr = (r // 128) * 128
                rr2 = (r2 // 128) * 128
                pltpu.async_copy(
                    src_t3_hbm.at[:, :, pl.ds(rr, 128)],
                    sblk_v.at[:, :, pl.ds(j * 128, 128)], sem)
                pltpu.async_copy(
                    dst_t3_hbm.at[:, :, pl.ds(rr2, 128)],
                    dblk_v.at[:, :, pl.ds(j * 128, 128)], sem)

            dummy = src_t3_hbm.at[:, :, pl.ds(0, _RND * 128)]
            pltpu.make_async_copy(dummy, sblk_v, sem).wait()
            pltpu.make_async_copy(dummy, dblk_v, sem).wait()

            for j in range(_RND):
                col = jnp.full((16,), j * 128, jnp.int32)
                sv = plsc.load_gather(sblk_v, [cb_v, s_v, col + qs[j]])
                dv = plsc.load_gather(dblk_v, [cb_v, s_v, col + qd[j]])
                prod_v[pl.ds(j * 16, 16)] = sv * dv

            acc = jnp.zeros((16,), jnp.float32)
            for c in range(16):
                acc = acc + plsc.load_gather(prod_v, [iota16 * 16 + c])
            prob = 1.0 / (1.0 + jnp.exp(-acc))
            out_v[pl.ds(base, 16)] = prob
            return 0

        lax.fori_loop(0, n_rounds, body, 0)

        pltpu.sync_copy(out_v, out_hbm.at[pl.ds(wid * b_per_w, b_per_w)])

    return run(src_idx, dst_idx, src_t3, dst_t3)


# speedup vs baseline: 1.0980x; 1.0980x over previous
import functools

import jax
import jax.numpy as jnp
from jax import lax
from jax.experimental import pallas as pl
from jax.experimental.pallas import tpu as pltpu
from jax.experimental.pallas import tpu_sc as plsc

_RND = 16


def kernel(pairs, init_emb, output_vecs):
    B = pairs.shape[0]
    V, D = init_emb.shape
    info = plsc.get_sparse_core_info()
    nc, ns = info.num_cores, info.num_subcores
    nw = nc * ns
    b_per_w = B // nw
    n_rounds = b_per_w // _RND

    src_t3 = init_emb.T.reshape(2, 8, V)
    dst_t3 = output_vecs.T.reshape(2, 8, V)

    src_idx = pairs[:, 0].astype(jnp.int32).reshape(nw, b_per_w)
    dst_idx = pairs[:, 1].astype(jnp.int32).reshape(nw, b_per_w)

    mesh = plsc.VectorSubcoreMesh(core_axis_name="c", subcore_axis_name="s")

    @functools.partial(
        pl.kernel,
        mesh=mesh,
        out_type=jax.ShapeDtypeStruct((B,), jnp.float32),
        compiler_params=pltpu.CompilerParams(needs_layout_passes=False),
        scratch_types=[
            pltpu.VMEM((b_per_w,), jnp.int32),
            pltpu.VMEM((b_per_w,), jnp.int32),
            pltpu.VMEM((2, 2, 8, _RND * 16), jnp.float32),
            pltpu.VMEM((2, 2, 8, _RND * 16), jnp.float32),
            pltpu.VMEM((_RND * 16,), jnp.float32),
            pltpu.VMEM((b_per_w,), jnp.float32),
            pltpu.SemaphoreType.DMA,
            pltpu.SemaphoreType.DMA,
        ],
    )
    def run(src_idx_hbm, dst_idx_hbm, src_t3_hbm, dst_t3_hbm, out_hbm,
            sidx_v, didx_v, sblk_v, dblk_v, prod_v, out_v, semA, semB):
        wid = lax.axis_index("s") * nc + lax.axis_index("c")

        pltpu.sync_copy(src_idx_hbm.at[wid], sidx_v)
        pltpu.sync_copy(dst_idx_hbm.at[wid], didx_v)

        iota16 = lax.iota(jnp.int32, 16)
        cb_v = iota16 // 8
        s_v = iota16 % 8

        def issue(rnd, p, sem):
            base = rnd * _RND
            for j in range(_RND):
                t, jj = j // 16, j % 16
                siv = sidx_v[pl.ds(base + t * 16, 16)]
                div = didx_v[pl.ds(base + t * 16, 16)]
                r = jnp.squeeze(lax.slice(siv, (jj,), (jj + 1,)))
                r2 = jnp.squeeze(lax.slice(div, (jj,), (jj + 1,)))
                rr = (r // 16) * 16
                rr2 = (r2 // 16) * 16
                pltpu.async_copy(
                    src_t3_hbm.at[:, :, pl.ds(rr, 16)],
                    sblk_v.at[p, :, :, pl.ds(j * 16, 16)], sem)
                pltpu.async_copy(
                    dst_t3_hbm.at[:, :, pl.ds(rr2, 16)],
                    dblk_v.at[p, :, :, pl.ds(j * 16, 16)], sem)

        def drain(p, sem):
            dummy = src_t3_hbm.at[:, :, pl.ds(0, _RND * 16)]
            pltpu.make_async_copy(dummy, sblk_v.at[p], sem).wait()
            pltpu.make_async_copy(dummy, dblk_v.at[p], sem).wait()

        def compute(rnd, p):
            base = rnd * _RND
            for j in range(_RND):
                t, jj = j // 16, j % 16
                siv = sidx_v[pl.ds(base + t * 16, 16)]
                div = didx_v[pl.ds(base + t * 16, 16)]
                q = jnp.squeeze(lax.slice(siv, (jj,), (jj + 1,))) % 16
                q2 = jnp.squeeze(lax.slice(div, (jj,), (jj + 1,))) % 16
                col = jnp.full((16,), j * 16, jnp.int32)
                sv = plsc.load_gather(sblk_v.at[p], [cb_v, s_v, col + q])
                dv = plsc.load_gather(dblk_v.at[p], [cb_v, s_v, col + q2])
                prod_v[pl.ds(j * 16, 16)] = sv * dv
            for t in range(_RND // 16):
                acc = jnp.zeros((16,), jnp.float32)
                for c in range(16):
                    acc = acc + plsc.load_gather(
                        prod_v, [iota16 * 16 + t * 256 + c])
                prob = 1.0 / (1.0 + jnp.exp(-acc))
                out_v[pl.ds(base + t * 16, 16)] = prob

        issue(0, 0, semA)

        def body(i, _):
            g = i * 2
            issue(g + 1, 1, semB)
            drain(0, semA)
            compute(g, 0)

            @pl.when(g + 2 < n_rounds)
            def _():
                issue(g + 2, 0, semA)

            drain(1, semB)
            compute(g + 1, 1)
            return 0

        lax.fori_loop(0, n_rounds // 2, body, 0)

        pltpu.sync_copy(out_v, out_hbm.at[pl.ds(wid * b_per_w, b_per_w)])

    return run(src_idx, dst_idx, src_t3, dst_t3)


# speedup vs baseline: 1.2655x; 1.1526x over previous
import functools

import jax
import jax.numpy as jnp
from jax import lax
from jax.experimental import pallas as pl
from jax.experimental.pallas import tpu as pltpu
from jax.experimental.pallas import tpu_sc as plsc

_RND = 16


def kernel(pairs, init_emb, output_vecs):
    B = pairs.shape[0]
    V, D = init_emb.shape
    info = plsc.get_sparse_core_info()
    nc, ns = info.num_cores, info.num_subcores
    nw = nc * ns
    b_per_w = B // nw
    n_rounds = b_per_w // _RND

    src_t3 = init_emb.T.reshape(2, 8, V)
    dst_t3 = output_vecs.T.reshape(2, 8, V)

    src_idx = pairs[:, 0].astype(jnp.int32).reshape(nw, b_per_w)
    dst_idx = pairs[:, 1].astype(jnp.int32).reshape(nw, b_per_w)

    mesh = plsc.VectorSubcoreMesh(core_axis_name="c", subcore_axis_name="s")

    @functools.partial(
        pl.kernel,
        mesh=mesh,
        out_type=jax.ShapeDtypeStruct((B,), jnp.float32),
        compiler_params=pltpu.CompilerParams(needs_layout_passes=False),
        scratch_types=[
            pltpu.VMEM((b_per_w,), jnp.int32),
            pltpu.VMEM((b_per_w,), jnp.int32),
            pltpu.VMEM((2, 2, 8, _RND * 16), jnp.float32),
            pltpu.VMEM((2, 2, 8, _RND * 16), jnp.float32),
            pltpu.VMEM((_RND * 16,), jnp.float32),
            pltpu.VMEM((b_per_w,), jnp.float32),
            pltpu.SemaphoreType.DMA,
            pltpu.SemaphoreType.DMA,
        ],
    )
    def run(src_idx_hbm, dst_idx_hbm, src_t3_hbm, dst_t3_hbm, out_hbm,
            sidx_v, didx_v, sblk_v, dblk_v, prod_v, out_v, semA, semB):
        wid = lax.axis_index("s") * nc + lax.axis_index("c")

        pltpu.sync_copy(src_idx_hbm.at[wid], sidx_v)
        pltpu.sync_copy(dst_idx_hbm.at[wid], didx_v)

        iota16 = lax.iota(jnp.int32, 16)
        cb_v = iota16 // 8
        s_v = iota16 % 8

        def issue(rnd, p, sem):
            base = rnd * _RND
            siv = sidx_v[pl.ds(base, _RND)]
            div = didx_v[pl.ds(base, _RND)]
            for j in range(_RND):
                r = jnp.squeeze(lax.slice(siv, (j,), (j + 1,)))
                r2 = jnp.squeeze(lax.slice(div, (j,), (j + 1,)))
                rr = (r // 16) * 16
                rr2 = (r2 // 16) * 16
                pltpu.async_copy(
                    src_t3_hbm.at[:, :, pl.ds(rr, 16)],
                    sblk_v.at[p, :, :, pl.ds(j * 16, 16)], sem)
                pltpu.async_copy(
                    dst_t3_hbm.at[:, :, pl.ds(rr2, 16)],
                    dblk_v.at[p, :, :, pl.ds(j * 16, 16)], sem)

        def drain(p, sem):
            dummy = src_t3_hbm.at[:, :, pl.ds(0, _RND * 16)]
            pltpu.make_async_copy(dummy, sblk_v.at[p], sem).wait()
            pltpu.make_async_copy(dummy, dblk_v.at[p], sem).wait()

        def compute(rnd, p):
            base = rnd * _RND
            siv = sidx_v[pl.ds(base, _RND)]
            div = didx_v[pl.ds(base, _RND)]
            for j in range(_RND):
                q = jnp.squeeze(lax.slice(siv, (j,), (j + 1,))) % 16
                q2 = jnp.squeeze(lax.slice(div, (j,), (j + 1,))) % 16
                col = jnp.full((16,), j * 16, jnp.int32)
                sv = plsc.load_gather(sblk_v.at[p], [cb_v, s_v, col + q])
                dv = plsc.load_gather(dblk_v.at[p], [cb_v, s_v, col + q2])
                prod_v[pl.ds(j * 16, 16)] = sv * dv
            acc = jnp.zeros((16,), jnp.float32)
            for c in range(16):
                acc = acc + plsc.load_gather(prod_v, [iota16 * 16 + c])
            prob = 1.0 / (1.0 + jnp.exp(-acc))
            out_v[pl.ds(base, 16)] = prob

        issue(0, 0, semA)

        def body(i, _):
            g = i * 2
            issue(g + 1, 1, semB)
            drain(0, semA)
            compute(g, 0)

            @pl.when(g + 2 < n_rounds)
            def _():
                issue(g + 2, 0, semA)

            drain(1, semB)
            compute(g + 1, 1)
            return 0

        lax.fori_loop(0, n_rounds // 2, body, 0)

        pltpu.sync_copy(out_v, out_hbm.at[pl.ds(wid * b_per_w, b_per_w)])

    return run(src_idx, dst_idx, src_t3, dst_t3)
